# baseline (speedup 1.0000x reference)
import functools

import jax
import jax.numpy as jnp
from jax import lax
from jax.experimental import pallas as pl
from jax.experimental.pallas import tpu as pltpu
from jax.experimental.pallas import tpu_sc as plsc

NUM_CORES = 2
NUM_SUBCORES = 16
NUM_WORKERS = NUM_CORES * NUM_SUBCORES
NBUF = 4
LOOKAHEAD = 2
L = 16


@functools.lru_cache(maxsize=None)
def _make_gather(batch: int, seq: int, dim: int):
    assert batch % NUM_WORKERS == 0
    b_per_w = batch // NUM_WORKERS
    assert seq == 200 and b_per_w == 128
    half = 128
    rem = seq - half
    mesh = plsc.VectorSubcoreMesh(core_axis_name="c", subcore_axis_name="s")

    def body(idxT_hbm, weight_hbm, out_hbm, idx_v, idx_t, rows, *sems):
        gsem = sems[:NBUF]
        wsem = sems[NBUF:]
        wid = lax.axis_index("s") * NUM_CORES + lax.axis_index("c")
        b0 = wid * b_per_w
        pltpu.sync_copy(idxT_hbm.at[:, pl.ds(b0 * 1, b_per_w)], idx_v)

        lanes = lax.iota(jnp.int32, L)

        @pl.loop(0, b_per_w)
        def _(b):
            for g in range(13):
                s_vec = jnp.minimum(g * L + lanes, seq - 1)
                vals = plsc.load_gather(
                    idx_v, [s_vec, jnp.broadcast_to(b, (L,)).astype(jnp.int32)])
                flat = g * L
                idx_t[b, flat // 128, pl.ds(flat % 128, L)] = vals

        def buf(k):
            return rows.at[k]

        def issue_gathers(b, k):
            pltpu.async_copy(weight_hbm.at[idx_t.at[b, 0]],
                             buf(k).at[pl.ds(0, half)], gsem[k])
            pltpu.async_copy(weight_hbm.at[idx_t.at[b, 1, pl.ds(0, rem)]],
                             buf(k).at[pl.ds(half, rem)], gsem[k])

        def wait_gathers(k):
            pltpu.make_async_copy(weight_hbm.at[idx_t.at[0, 0]],
                                  buf(k).at[pl.ds(0, half)], gsem[k]).wait()
            pltpu.make_async_copy(weight_hbm.at[idx_t.at[0, 1, pl.ds(0, rem)]],
                                  buf(k).at[pl.ds(half, rem)], gsem[k]).wait()

        def issue_write(b, k):
            pltpu.async_copy(buf(k), out_hbm.at[b0 + b], wsem[k])

        def wait_write(k):
            pltpu.make_async_copy(buf(k), out_hbm.at[b0], wsem[k]).wait()

        for b in range(LOOKAHEAD):
            issue_gathers(b, b % NBUF)

        @pl.loop(0, b_per_w, step=NBUF)
        def _(i0):
            for j in range(NBUF):
                i = i0 + j
                ka = (j + LOOKAHEAD) % NBUF

                @pl.when(i + LOOKAHEAD < b_per_w)
                def _():
                    @pl.when(i + LOOKAHEAD >= NBUF)
                    def _():
                        wait_write(ka)
                    issue_gathers(i + LOOKAHEAD, ka)

                wait_gathers(j)
                issue_write(i, j)

        for k in range(NBUF):
            wait_write(k)

    return pl.kernel(
        body,
        out_type=jax.ShapeDtypeStruct((batch, seq, dim), jnp.float32),
        mesh=mesh,
        scratch_types=[
            pltpu.VMEM((seq, b_per_w), jnp.int32),
            pltpu.VMEM((b_per_w, 2, 128), jnp.int32),
            pltpu.VMEM((NBUF, seq, dim), jnp.float32),
        ] + [pltpu.SemaphoreType.DMA] * (2 * NBUF),
        compiler_params=pltpu.CompilerParams(
            use_tc_tiling_on_sc=False, needs_layout_passes=False),
    )


def kernel(token_ids, weight):
    batch, seq = token_ids.shape
    _, dim = weight.shape
    idxT = token_ids.T.astype(jnp.int32)
    return _make_gather(batch, seq, dim)(idxT, weight)
